# baseline (speedup 1.0000x reference)
import jax
import jax.numpy as jnp
from jax import lax
from jax.experimental import pallas as pl
from jax.experimental.pallas import tpu as pltpu
from jax.experimental.pallas import tpu_sc as plsc

B = 16384
D = 24
NC = 2
NS = 16
NW = NC * NS
BPW = B // NW
L = 16


def _sc_body(weeks_hbm, seasons_hbm, hol_hbm, w_hbm,
             wk_idx_hbm, dy_idx_hbm, hl_idx_hbm,
             out_hbm,
             weeks_v, seasons_v, hol_v,
             wk_v, dy_v, hl_v, out_v, w_v, sem):
    wid = lax.axis_index("s") * NC + lax.axis_index("c")
    base = wid * BPW

    pltpu.sync_copy(weeks_hbm, weeks_v)
    pltpu.sync_copy(seasons_hbm, seasons_v)
    pltpu.sync_copy(hol_hbm, hol_v)
    pltpu.sync_copy(w_hbm, w_v)
    pltpu.sync_copy(wk_idx_hbm.at[pl.ds(base, BPW)], wk_v)
    pltpu.sync_copy(dy_idx_hbm.at[pl.ds(base, BPW)], dy_v)
    pltpu.sync_copy(hl_idx_hbm.at[pl.ds(base, BPW)], hl_v)

    wv = w_v[pl.ds(0, L)]
    w1 = jnp.full((L,), wv[0], jnp.float32)
    w2 = jnp.full((L,), wv[1], jnp.float32)
    w3 = jnp.full((L,), wv[2], jnp.float32)
    lane = lax.iota(jnp.int32, L)

    def group(g, carry):
        b0 = g * L
        wk = wk_v[pl.ds(b0, L)]
        dy = dy_v[pl.ds(b0, L)]
        hl = hl_v[pl.ds(b0, L)]
        rows = b0 + lane
        for d in range(D):
            dcol = jnp.full((L,), d, jnp.int32)
            a = plsc.load_gather(weeks_v, [wk, dcol])
            b = plsc.load_gather(seasons_v, [dy, dcol])
            c = plsc.load_gather(hol_v, [hl, dcol])
            val = w1 * a + w2 * b + w3 * c
            plsc.store_scatter(out_v, [rows, dcol], val)
        return carry

    lax.fori_loop(0, BPW // L, group, 0)

    pltpu.sync_copy(out_v, out_hbm.at[pl.ds(base, BPW)])


def kernel(weeks, seasons, holidays_tab, w1, w2, w3, week_idx, day_idx, holiday_idx):
    w = jnp.pad(jnp.stack([w1, w2, w3]), (0, L - 3))
    mesh = plsc.VectorSubcoreMesh(core_axis_name="c", subcore_axis_name="s")
    f = pl.kernel(
        _sc_body,
        mesh=mesh,
        compiler_params=pltpu.CompilerParams(needs_layout_passes=False),
        out_type=jax.ShapeDtypeStruct((B, D), jnp.float32),
        scratch_types=[
            pltpu.VMEM((53, D), jnp.float32),
            pltpu.VMEM((7, D), jnp.float32),
            pltpu.VMEM((2, D), jnp.float32),
            pltpu.VMEM((BPW,), jnp.int32),
            pltpu.VMEM((BPW,), jnp.int32),
            pltpu.VMEM((BPW,), jnp.int32),
            pltpu.VMEM((BPW, D), jnp.float32),
            pltpu.VMEM((L,), jnp.float32),
            pltpu.SemaphoreType.DMA,
        ],
    )
    return f(weeks, seasons, holidays_tab, w, week_idx, day_idx, holiday_idx)


# speedup vs baseline: 1.2278x; 1.2278x over previous
import jax
import jax.numpy as jnp
from jax import lax
from jax.experimental import pallas as pl
from jax.experimental.pallas import tpu as pltpu
from jax.experimental.pallas import tpu_sc as plsc

B = 16384
D = 24
DP = 128
NC = 2
NS = 16
NW = NC * NS
BPW = B // NW
L = 16


def _sc_body(weeks_hbm, seasons_hbm, hol_hbm, w_hbm,
             wk_idx_hbm, dy_idx_hbm, hl_idx_hbm,
             out_hbm,
             weeks_v, seasons_v, hol_v,
             wk_v, dy_v, hl_v, out_v, w_v, sem):
    wid = lax.axis_index("s") * NC + lax.axis_index("c")
    base = wid * BPW

    pltpu.sync_copy(weeks_hbm, weeks_v)
    pltpu.sync_copy(seasons_hbm, seasons_v)
    pltpu.sync_copy(hol_hbm, hol_v)
    pltpu.sync_copy(w_hbm, w_v)
    pltpu.sync_copy(wk_idx_hbm.at[pl.ds(base, BPW)], wk_v)
    pltpu.sync_copy(dy_idx_hbm.at[pl.ds(base, BPW)], dy_v)
    pltpu.sync_copy(hl_idx_hbm.at[pl.ds(base, BPW)], hl_v)

    wv = w_v[pl.ds(0, L)]
    w1 = jnp.full((L,), wv[0], jnp.float32)
    w2 = jnp.full((L,), wv[1], jnp.float32)
    w3 = jnp.full((L,), wv[2], jnp.float32)
    lane = lax.iota(jnp.int32, L)

    def group(g, carry):
        b0 = g * L
        wk = wk_v[pl.ds(b0, L)] * D
        dy = dy_v[pl.ds(b0, L)] * D
        hl = hl_v[pl.ds(b0, L)] * D
        rows = b0 + lane
        for d in range(D):
            a = plsc.load_gather(weeks_v, [wk + d])
            b = plsc.load_gather(seasons_v, [dy + d])
            c = plsc.load_gather(hol_v, [hl + d])
            val = w1 * a + w2 * b + w3 * c
            plsc.store_scatter(out_v, [rows, jnp.full((L,), d, jnp.int32)], val)
        return carry

    lax.fori_loop(0, BPW // L, group, 0)

    pltpu.sync_copy(out_v, out_hbm.at[pl.ds(base, BPW)])


def kernel(weeks, seasons, holidays_tab, w1, w2, w3, week_idx, day_idx, holiday_idx):
    w = jnp.pad(jnp.stack([w1, w2, w3]), (0, L - 3))
    mesh = plsc.VectorSubcoreMesh(core_axis_name="c", subcore_axis_name="s")
    f = pl.kernel(
        _sc_body,
        mesh=mesh,
        compiler_params=pltpu.CompilerParams(needs_layout_passes=False),
        out_type=jax.ShapeDtypeStruct((B, DP), jnp.float32),
        scratch_types=[
            pltpu.VMEM((53 * D,), jnp.float32),
            pltpu.VMEM((7 * D,), jnp.float32),
            pltpu.VMEM((2 * D,), jnp.float32),
            pltpu.VMEM((BPW,), jnp.int32),
            pltpu.VMEM((BPW,), jnp.int32),
            pltpu.VMEM((BPW,), jnp.int32),
            pltpu.VMEM((BPW, DP), jnp.float32),
            pltpu.VMEM((L,), jnp.float32),
            pltpu.SemaphoreType.DMA,
        ],
    )
    padded = f(weeks.reshape(-1), seasons.reshape(-1), holidays_tab.reshape(-1), w,
               week_idx, day_idx, holiday_idx)
    return padded[:, :D]


# speedup vs baseline: 1.3790x; 1.1231x over previous
import jax
import jax.numpy as jnp
from jax import lax
from jax.experimental import pallas as pl
from jax.experimental.pallas import tpu as pltpu
from jax.experimental.pallas import tpu_sc as plsc

B = 16384
D = 24
DP = 128
NC = 2
NS = 16
NW = NC * NS
BPW = B // NW
L = 16
NCHUNK = 4
CROWS = BPW // NCHUNK
GPC = CROWS // L

W_WORDS = 53 * D
S_WORDS = 7 * D
H_WORDS = 2 * D
C_WORDS = 14 * D


def _scale_into(src, dst, n_words, wvec):
    nfull = n_words // L
    for i in range(nfull):
        dst[pl.ds(i * L, L)] = wvec * src[pl.ds(i * L, L)]
    if n_words % L:
        off = n_words - L
        dst[pl.ds(off, L)] = wvec * src[pl.ds(off, L)]


def _sc_body(weeks_hbm, seasons_hbm, hol_hbm, w_hbm,
             wk_idx_hbm, dy_idx_hbm, hl_idx_hbm,
             out_hbm,
             weeks_v, seasons_v, hol_v,
             wsc_v, ssc_v, hsc_v, comb_v,
             wk_v, dy_v, hl_v, out_v, w_v, sem):
    wid = lax.axis_index("s") * NC + lax.axis_index("c")
    base = wid * BPW

    cps = [
        pltpu.make_async_copy(weeks_hbm, weeks_v, sem),
        pltpu.make_async_copy(seasons_hbm, seasons_v, sem),
        pltpu.make_async_copy(hol_hbm, hol_v, sem),
        pltpu.make_async_copy(w_hbm, w_v, sem),
        pltpu.make_async_copy(wk_idx_hbm.at[pl.ds(base, BPW)], wk_v, sem),
        pltpu.make_async_copy(dy_idx_hbm.at[pl.ds(base, BPW)], dy_v, sem),
        pltpu.make_async_copy(hl_idx_hbm.at[pl.ds(base, BPW)], hl_v, sem),
    ]
    for cp in cps:
        cp.start()
    for cp in cps:
        cp.wait()

    wv = w_v[pl.ds(0, L)]
    w1 = jnp.full((L,), wv[0], jnp.float32)
    w2 = jnp.full((L,), wv[1], jnp.float32)
    w3 = jnp.full((L,), wv[2], jnp.float32)
    lane = lax.iota(jnp.int32, L)

    _scale_into(weeks_v, wsc_v, W_WORDS, w1)
    _scale_into(seasons_v, ssc_v, S_WORDS, w2)
    _scale_into(hol_v, hsc_v, H_WORDS, w3)

    for s in range(7):
        for h in range(2):
            r = (s * 2 + h) * D
            for off in (0, 8):
                comb_v[pl.ds(r + off, L)] = (
                    ssc_v[pl.ds(s * D + off, L)] + hsc_v[pl.ds(h * D + off, L)]
                )

    def group(g, carry):
        b0 = g * L
        wk = wk_v[pl.ds(b0, L)] * D
        cb = dy_v[pl.ds(b0, L)] * (2 * D) + hl_v[pl.ds(b0, L)] * D
        rows = b0 + lane
        for d in range(D):
            a = plsc.load_gather(wsc_v, [wk + d])
            b = plsc.load_gather(comb_v, [cb + d])
            plsc.store_scatter(out_v, [rows, jnp.full((L,), d, jnp.int32)], a + b)
        return carry

    out_cps = []
    for c in range(NCHUNK):
        lax.fori_loop(c * GPC, (c + 1) * GPC, group, 0)
        cp = pltpu.make_async_copy(
            out_v.at[pl.ds(c * CROWS, CROWS)],
            out_hbm.at[pl.ds(base + c * CROWS, CROWS)],
            sem,
        )
        cp.start()
        out_cps.append(cp)
    for cp in out_cps:
        cp.wait()


def kernel(weeks, seasons, holidays_tab, w1, w2, w3, week_idx, day_idx, holiday_idx):
    w = jnp.pad(jnp.stack([w1, w2, w3]), (0, L - 3))
    mesh = plsc.VectorSubcoreMesh(core_axis_name="c", subcore_axis_name="s")
    f = pl.kernel(
        _sc_body,
        mesh=mesh,
        compiler_params=pltpu.CompilerParams(needs_layout_passes=False),
        out_type=jax.ShapeDtypeStruct((B, DP), jnp.float32),
        scratch_types=[
            pltpu.VMEM((W_WORDS,), jnp.float32),
            pltpu.VMEM((S_WORDS,), jnp.float32),
            pltpu.VMEM((H_WORDS,), jnp.float32),
            pltpu.VMEM((W_WORDS + 8,), jnp.float32),
            pltpu.VMEM((S_WORDS + 8,), jnp.float32),
            pltpu.VMEM((H_WORDS,), jnp.float32),
            pltpu.VMEM((C_WORDS,), jnp.float32),
            pltpu.VMEM((BPW,), jnp.int32),
            pltpu.VMEM((BPW,), jnp.int32),
            pltpu.VMEM((BPW,), jnp.int32),
            pltpu.VMEM((BPW, DP), jnp.float32),
            pltpu.VMEM((L,), jnp.float32),
            pltpu.SemaphoreType.DMA,
        ],
    )
    padded = f(weeks.reshape(-1), seasons.reshape(-1), holidays_tab.reshape(-1), w,
               week_idx, day_idx, holiday_idx)
    return padded[:, :D]


# speedup vs baseline: 1.6535x; 1.1991x over previous
import jax
import jax.numpy as jnp
from jax import lax
from jax.experimental import pallas as pl
from jax.experimental.pallas import tpu as pltpu
from jax.experimental.pallas import tpu_sc as plsc

B = 16384
D = 24
DS = 25
DP = 128
DQ = 129
NC = 2
NS = 16
NW = NC * NS
BPW = B // NW
L = 16
NCHUNK = 4
CROWS = BPW // NCHUNK
GPC = CROWS // L

W_WORDS = 53 * D
S_WORDS = 7 * D
H_WORDS = 2 * D


def _sc_body(weeks_hbm, seasons_hbm, hol_hbm, w_hbm,
             wk_idx_hbm, dy_idx_hbm, hl_idx_hbm,
             out_hbm,
             weeks_v, seasons_v, hol_v,
             wsc_v, comb_v,
             wk_v, dy_v, hl_v, stage_v, w_v, sem):
    wid = lax.axis_index("s") * NC + lax.axis_index("c")
    base = wid * BPW

    cps = [
        pltpu.make_async_copy(weeks_hbm, weeks_v, sem),
        pltpu.make_async_copy(seasons_hbm, seasons_v, sem),
        pltpu.make_async_copy(hol_hbm, hol_v, sem),
        pltpu.make_async_copy(w_hbm, w_v, sem),
        pltpu.make_async_copy(wk_idx_hbm.at[pl.ds(base, BPW)], wk_v, sem),
        pltpu.make_async_copy(dy_idx_hbm.at[pl.ds(base, BPW)], dy_v, sem),
        pltpu.make_async_copy(hl_idx_hbm.at[pl.ds(base, BPW)], hl_v, sem),
    ]
    for cp in cps:
        cp.start()
    for cp in cps:
        cp.wait()

    wv = w_v[pl.ds(0, L)]
    w1 = jnp.full((L,), wv[0], jnp.float32)
    w2 = jnp.full((L,), wv[1], jnp.float32)
    w3 = jnp.full((L,), wv[2], jnp.float32)
    lane = lax.iota(jnp.int32, L)

    for r in range(53):
        for off in (0, 8):
            wsc_v[pl.ds(r * DS + off, L)] = w1 * weeks_v[pl.ds(r * D + off, L)]

    for s in range(7):
        for h in range(2):
            r = (s * 2 + h) * DS
            for off in (0, 8):
                comb_v[pl.ds(r + off, L)] = (
                    w2 * seasons_v[pl.ds(s * D + off, L)]
                    + w3 * hol_v[pl.ds(h * D + off, L)]
                )

    dcols = []
    for t in range(D):
        x = lane + t
        dcols.append(jnp.where(x >= D, x - D, x))

    def group(g, carry):
        b0 = g * L
        wk = wk_v[pl.ds(b0, L)] * DS
        cb = dy_v[pl.ds(b0, L)] * (2 * DS) + hl_v[pl.ds(b0, L)] * DS
        rows = b0 + lane
        for t in range(D):
            dcol = dcols[t]
            a = plsc.load_gather(wsc_v, [wk + dcol])
            b = plsc.load_gather(comb_v, [cb + dcol])
            plsc.store_scatter(stage_v, [rows, dcol], a + b)
        return carry

    out_cps = []
    for c in range(NCHUNK):
        lax.fori_loop(c * GPC, (c + 1) * GPC, group, 0)
        cp = pltpu.make_async_copy(
            stage_v.at[pl.ds(c * CROWS, CROWS)],
            out_hbm.at[pl.ds(base + c * CROWS, CROWS)],
            sem,
        )
        cp.start()
        out_cps.append(cp)
    for cp in out_cps:
        cp.wait()


def kernel(weeks, seasons, holidays_tab, w1, w2, w3, week_idx, day_idx, holiday_idx):
    w = jnp.pad(jnp.stack([w1, w2, w3]), (0, L - 3))
    mesh = plsc.VectorSubcoreMesh(core_axis_name="c", subcore_axis_name="s")
    f = pl.kernel(
        _sc_body,
        mesh=mesh,
        compiler_params=pltpu.CompilerParams(needs_layout_passes=False),
        out_type=jax.ShapeDtypeStruct((B, DP), jnp.float32),
        scratch_types=[
            pltpu.VMEM((W_WORDS,), jnp.float32),
            pltpu.VMEM((S_WORDS,), jnp.float32),
            pltpu.VMEM((H_WORDS,), jnp.float32),
            pltpu.VMEM((53 * DS + 8,), jnp.float32),
            pltpu.VMEM((14 * DS + 8,), jnp.float32),
            pltpu.VMEM((BPW,), jnp.int32),
            pltpu.VMEM((BPW,), jnp.int32),
            pltpu.VMEM((BPW,), jnp.int32),
            pltpu.VMEM((BPW, DP), jnp.float32),
            pltpu.VMEM((L,), jnp.float32),
            pltpu.SemaphoreType.DMA,
        ],
    )
    padded = f(weeks.reshape(-1), seasons.reshape(-1), holidays_tab.reshape(-1), w,
               week_idx, day_idx, holiday_idx)
    return padded[:, :D]


# speedup vs baseline: 1.8775x; 1.1355x over previous
import jax
import jax.numpy as jnp
from jax import lax
from jax.experimental import pallas as pl
from jax.experimental.pallas import tpu as pltpu
from jax.experimental.pallas import tpu_sc as plsc

B = 16384
D = 24
DS = 25
DP = 128
NC = 2
NS = 16
NW = NC * NS
BPW = B // NW
L = 16
NCHUNK = 4
CROWS = BPW // NCHUNK
GPC = CROWS // L


def _sc_body(weeks_hbm, seasons_hbm, hol_hbm, w_hbm,
             wk_idx_hbm, dy_idx_hbm, hl_idx_hbm,
             out_hbm,
             weeks_v, seasons_v, hol_v,
             wsc_v, comb_v,
             wk_v, dy_v, hl_v, stage_v, w_v, sem):
    wid = lax.axis_index("s") * NC + lax.axis_index("c")
    base = wid * BPW

    cps = [
        pltpu.make_async_copy(weeks_hbm, weeks_v, sem),
        pltpu.make_async_copy(seasons_hbm, seasons_v, sem),
        pltpu.make_async_copy(hol_hbm, hol_v, sem),
        pltpu.make_async_copy(w_hbm, w_v.at[pl.ds(0, 3)], sem),
        pltpu.make_async_copy(wk_idx_hbm.at[pl.ds(base, BPW)], wk_v, sem),
        pltpu.make_async_copy(dy_idx_hbm.at[pl.ds(base, BPW)], dy_v, sem),
        pltpu.make_async_copy(hl_idx_hbm.at[pl.ds(base, BPW)], hl_v, sem),
    ]
    for cp in cps:
        cp.start()
    for cp in cps:
        cp.wait()

    wv = w_v[pl.ds(0, L)]
    w1 = jnp.full((L,), wv[0], jnp.float32)
    w2 = jnp.full((L,), wv[1], jnp.float32)
    w3 = jnp.full((L,), wv[2], jnp.float32)
    lane = lax.iota(jnp.int32, L)

    for r in range(53):
        for off in (0, 8):
            wsc_v[pl.ds(r * DS + off, L)] = w1 * weeks_v[r, pl.ds(off, L)]

    for s in range(7):
        for h in range(2):
            r = (s * 2 + h) * DS
            for off in (0, 8):
                comb_v[pl.ds(r + off, L)] = (
                    w2 * seasons_v[s, pl.ds(off, L)]
                    + w3 * hol_v[h, pl.ds(off, L)]
                )

    dcols = []
    for t in range(D):
        x = lane + t
        dcols.append(jnp.where(x >= D, x - D, x))

    def group(g, carry):
        b0 = g * L
        wk = wk_v[pl.ds(b0, L)] * DS
        cb = dy_v[pl.ds(b0, L)] * (2 * DS) + hl_v[pl.ds(b0, L)] * DS
        rows = b0 + lane
        for t in range(D // 2):
            tA, tB = t, t + D // 2
            aA = plsc.load_gather(wsc_v, [wk + dcols[tA]])
            bA = plsc.load_gather(comb_v, [cb + dcols[tA]])
            aB = plsc.load_gather(wsc_v, [wk + dcols[tB]])
            bB = plsc.load_gather(comb_v, [cb + dcols[tB]])
            plsc.store_scatter(stage_v, [rows, dcols[tA]], aA + bA)
            plsc.store_scatter(stage_v, [rows, dcols[tB]], aB + bB)
        return carry

    out_cps = []
    for c in range(NCHUNK):
        lax.fori_loop(c * GPC, (c + 1) * GPC, group, 0)
        cp = pltpu.make_async_copy(
            stage_v.at[pl.ds(c * CROWS, CROWS)],
            out_hbm.at[pl.ds(base + c * CROWS, CROWS)],
            sem,
        )
        cp.start()
        out_cps.append(cp)
    for cp in out_cps:
        cp.wait()


def kernel(weeks, seasons, holidays_tab, w1, w2, w3, week_idx, day_idx, holiday_idx):
    w = jnp.stack([w1, w2, w3])
    mesh = plsc.VectorSubcoreMesh(core_axis_name="c", subcore_axis_name="s")
    f = pl.kernel(
        _sc_body,
        mesh=mesh,
        compiler_params=pltpu.CompilerParams(needs_layout_passes=False),
        out_type=jax.ShapeDtypeStruct((B, DP), jnp.float32),
        scratch_types=[
            pltpu.VMEM((53, D), jnp.float32),
            pltpu.VMEM((7, D), jnp.float32),
            pltpu.VMEM((2, D), jnp.float32),
            pltpu.VMEM((53 * DS + 8,), jnp.float32),
            pltpu.VMEM((14 * DS + 8,), jnp.float32),
            pltpu.VMEM((BPW,), jnp.int32),
            pltpu.VMEM((BPW,), jnp.int32),
            pltpu.VMEM((BPW,), jnp.int32),
            pltpu.VMEM((BPW, DP), jnp.float32),
            pltpu.VMEM((L,), jnp.float32),
            pltpu.SemaphoreType.DMA,
        ],
    )
    padded = f(weeks, seasons, holidays_tab, w,
               week_idx, day_idx, holiday_idx)
    return padded[:, :D]


# speedup vs baseline: 1.9252x; 1.0254x over previous
import jax
import jax.numpy as jnp
from jax import lax
from jax.experimental import pallas as pl
from jax.experimental.pallas import tpu as pltpu
from jax.experimental.pallas import tpu_sc as plsc

B = 16384
D = 24
DS = 25
DP = 128
NC = 2
NS = 16
NW = NC * NS
BPW = B // NW
L = 16
NCHUNK = 4
CROWS = BPW // NCHUNK
GPC = CROWS // L


def _sc_body(weeks_hbm, seasons_hbm, hol_hbm, w_hbm,
             wk_idx_hbm, dy_idx_hbm, hl_idx_hbm,
             out_hbm,
             weeks_v, seasons_v, hol_v,
             wsc_v, comb_v,
             wk_v, dy_v, hl_v, stage_v, w_v, sem):
    wid = lax.axis_index("s") * NC + lax.axis_index("c")
    base = wid * BPW

    cps = [
        pltpu.make_async_copy(weeks_hbm, weeks_v, sem),
        pltpu.make_async_copy(seasons_hbm, seasons_v, sem),
        pltpu.make_async_copy(hol_hbm, hol_v, sem),
        pltpu.make_async_copy(w_hbm, w_v.at[pl.ds(0, 3)], sem),
        pltpu.make_async_copy(wk_idx_hbm.at[pl.ds(base, BPW)], wk_v, sem),
        pltpu.make_async_copy(dy_idx_hbm.at[pl.ds(base, BPW)], dy_v, sem),
        pltpu.make_async_copy(hl_idx_hbm.at[pl.ds(base, BPW)], hl_v, sem),
    ]
    for cp in cps:
        cp.start()
    for cp in cps:
        cp.wait()

    wv = w_v[pl.ds(0, L)]
    w1 = jnp.full((L,), wv[0], jnp.float32)
    w2 = jnp.full((L,), wv[1], jnp.float32)
    w3 = jnp.full((L,), wv[2], jnp.float32)
    lane = lax.iota(jnp.int32, L)

    for r in range(53):
        for off in (0, 8):
            wsc_v[pl.ds(r * DS + off, L)] = w1 * weeks_v[r, pl.ds(off, L)]

    for s in range(7):
        for h in range(2):
            r = (s * 2 + h) * DS
            for off in (0, 8):
                comb_v[pl.ds(r + off, L)] = (
                    w2 * seasons_v[s, pl.ds(off, L)]
                    + w3 * hol_v[h, pl.ds(off, L)]
                )

    dcols = []
    for t in range(D):
        x = lane + t
        dcols.append(jnp.where(x >= D, x - D, x))

    def group(g, carry):
        b0 = g * L
        wk = wk_v[pl.ds(b0, L)] * DS
        cb = dy_v[pl.ds(b0, L)] * (2 * DS) + hl_v[pl.ds(b0, L)] * DS
        rows = b0 + lane
        for t in range(D // 3):
            ts = (t, t + D // 3, t + 2 * (D // 3))
            ab = [(plsc.load_gather(wsc_v, [wk + dcols[tt]]),
                   plsc.load_gather(comb_v, [cb + dcols[tt]])) for tt in ts]
            for tt, (a, b) in zip(ts, ab):
                plsc.store_scatter(stage_v, [rows, dcols[tt]], a + b)
        return carry

    out_cps = []
    for c in range(NCHUNK):
        lax.fori_loop(c * GPC, (c + 1) * GPC, group, 0)
        cp = pltpu.make_async_copy(
            stage_v.at[pl.ds(c * CROWS, CROWS)],
            out_hbm.at[pl.ds(base + c * CROWS, CROWS)],
            sem,
        )
        cp.start()
        out_cps.append(cp)
    for cp in out_cps:
        cp.wait()


def kernel(weeks, seasons, holidays_tab, w1, w2, w3, week_idx, day_idx, holiday_idx):
    w = jnp.stack([w1, w2, w3])
    mesh = plsc.VectorSubcoreMesh(core_axis_name="c", subcore_axis_name="s")
    f = pl.kernel(
        _sc_body,
        mesh=mesh,
        compiler_params=pltpu.CompilerParams(needs_layout_passes=False),
        out_type=jax.ShapeDtypeStruct((B, DP), jnp.float32),
        scratch_types=[
            pltpu.VMEM((53, D), jnp.float32),
            pltpu.VMEM((7, D), jnp.float32),
            pltpu.VMEM((2, D), jnp.float32),
            pltpu.VMEM((53 * DS + 8,), jnp.float32),
            pltpu.VMEM((14 * DS + 8,), jnp.float32),
            pltpu.VMEM((BPW,), jnp.int32),
            pltpu.VMEM((BPW,), jnp.int32),
            pltpu.VMEM((BPW,), jnp.int32),
            pltpu.VMEM((BPW, DP), jnp.float32),
            pltpu.VMEM((L,), jnp.float32),
            pltpu.SemaphoreType.DMA,
        ],
    )
    padded = f(weeks, seasons, holidays_tab, w,
               week_idx, day_idx, holiday_idx)
    return padded[:, :D]


# speedup vs baseline: 1.9323x; 1.0037x over previous
import jax
import jax.numpy as jnp
from jax import lax
from jax.experimental import pallas as pl
from jax.experimental.pallas import tpu as pltpu
from jax.experimental.pallas import tpu_sc as plsc

B = 16384
D = 24
DS = 25
DP = 128
NC = 2
NS = 16
NW = NC * NS
BPW = B // NW
L = 16
NCHUNK = 4
CROWS = BPW // NCHUNK
GPC = CROWS // L


def _sc_body(weeks_hbm, seasons_hbm, hol_hbm, w_hbm,
             wk_idx_hbm, dy_idx_hbm, hl_idx_hbm,
             out_hbm,
             weeks_v, seasons_v, hol_v,
             wsc_v, comb_v,
             wk_v, dy_v, hl_v, stage_v, w_v, sem):
    wid = lax.axis_index("s") * NC + lax.axis_index("c")
    base = wid * BPW

    cps = [
        pltpu.make_async_copy(weeks_hbm, weeks_v, sem),
        pltpu.make_async_copy(seasons_hbm, seasons_v, sem),
        pltpu.make_async_copy(hol_hbm, hol_v, sem),
        pltpu.make_async_copy(w_hbm, w_v.at[pl.ds(0, 3)], sem),
        pltpu.make_async_copy(wk_idx_hbm.at[pl.ds(base, BPW)], wk_v, sem),
        pltpu.make_async_copy(dy_idx_hbm.at[pl.ds(base, BPW)], dy_v, sem),
        pltpu.make_async_copy(hl_idx_hbm.at[pl.ds(base, BPW)], hl_v, sem),
    ]
    for cp in cps:
        cp.start()
    for cp in cps:
        cp.wait()

    wv = w_v[pl.ds(0, L)]
    w1 = jnp.full((L,), wv[0], jnp.float32)
    w2 = jnp.full((L,), wv[1], jnp.float32)
    w3 = jnp.full((L,), wv[2], jnp.float32)
    lane = lax.iota(jnp.int32, L)

    for r in range(53):
        for off in (0, 8):
            wsc_v[pl.ds(r * DS + off, L)] = w1 * weeks_v[r, pl.ds(off, L)]

    for s in range(7):
        for h in range(2):
            r = (s * 2 + h) * DS
            for off in (0, 8):
                comb_v[pl.ds(r + off, L)] = (
                    w2 * seasons_v[s, pl.ds(off, L)]
                    + w3 * hol_v[h, pl.ds(off, L)]
                )

    dcols = []
    for t in range(D):
        x = lane + t
        dcols.append(jnp.where(x >= D, x - D, x))

    def group(g, carry):
        b0 = g * L
        wk = wk_v[pl.ds(b0, L)] * DS
        cb = dy_v[pl.ds(b0, L)] * (2 * DS) + hl_v[pl.ds(b0, L)] * DS
        rows = b0 + lane
        for t in range(D // 4):
            ts = (t, t + D // 4, t + 2 * (D // 4), t + 3 * (D // 4))
            ab = [(plsc.load_gather(wsc_v, [wk + dcols[tt]]),
                   plsc.load_gather(comb_v, [cb + dcols[tt]])) for tt in ts]
            for tt, (a, b) in zip(ts, ab):
                plsc.store_scatter(stage_v, [rows, dcols[tt]], a + b)
        return carry

    out_cps = []
    for c in range(NCHUNK):
        lax.fori_loop(c * GPC, (c + 1) * GPC, group, 0)
        cp = pltpu.make_async_copy(
            stage_v.at[pl.ds(c * CROWS, CROWS)],
            out_hbm.at[pl.ds(base + c * CROWS, CROWS)],
            sem,
        )
        cp.start()
        out_cps.append(cp)
    for cp in out_cps:
        cp.wait()


def kernel(weeks, seasons, holidays_tab, w1, w2, w3, week_idx, day_idx, holiday_idx):
    w = jnp.stack([w1, w2, w3])
    mesh = plsc.VectorSubcoreMesh(core_axis_name="c", subcore_axis_name="s")
    f = pl.kernel(
        _sc_body,
        mesh=mesh,
        compiler_params=pltpu.CompilerParams(needs_layout_passes=False),
        out_type=jax.ShapeDtypeStruct((B, DP), jnp.float32),
        scratch_types=[
            pltpu.VMEM((53, D), jnp.float32),
            pltpu.VMEM((7, D), jnp.float32),
            pltpu.VMEM((2, D), jnp.float32),
            pltpu.VMEM((53 * DS + 8,), jnp.float32),
            pltpu.VMEM((14 * DS + 8,), jnp.float32),
            pltpu.VMEM((BPW,), jnp.int32),
            pltpu.VMEM((BPW,), jnp.int32),
            pltpu.VMEM((BPW,), jnp.int32),
            pltpu.VMEM((BPW, DP), jnp.float32),
            pltpu.VMEM((L,), jnp.float32),
            pltpu.SemaphoreType.DMA,
        ],
    )
    padded = f(weeks, seasons, holidays_tab, w,
               week_idx, day_idx, holiday_idx)
    return padded[:, :D]


# speedup vs baseline: 1.9397x; 1.0038x over previous
import jax
import jax.numpy as jnp
from jax import lax
from jax.experimental import pallas as pl
from jax.experimental.pallas import tpu as pltpu
from jax.experimental.pallas import tpu_sc as plsc

B = 16384
D = 24
DS = 25
DP = 128
NC = 2
NS = 16
NW = NC * NS
BPW = B // NW
L = 16
NCHUNK = 4
CROWS = BPW // NCHUNK
GPC = CROWS // L


def _sc_body(weeks_hbm, seasons_hbm, hol_hbm, w_hbm,
             wk_idx_hbm, dy_idx_hbm, hl_idx_hbm,
             out_hbm,
             weeks_v, seasons_v, hol_v,
             wsc_v, comb_v,
             wk_v, dy_v, hl_v, stage_v, w_v, sem):
    wid = lax.axis_index("s") * NC + lax.axis_index("c")
    base = wid * BPW

    cps = [
        pltpu.make_async_copy(weeks_hbm, weeks_v, sem),
        pltpu.make_async_copy(seasons_hbm, seasons_v, sem),
        pltpu.make_async_copy(hol_hbm, hol_v, sem),
        pltpu.make_async_copy(w_hbm, w_v.at[pl.ds(0, 3)], sem),
        pltpu.make_async_copy(wk_idx_hbm.at[pl.ds(base, BPW)], wk_v, sem),
        pltpu.make_async_copy(dy_idx_hbm.at[pl.ds(base, BPW)], dy_v, sem),
        pltpu.make_async_copy(hl_idx_hbm.at[pl.ds(base, BPW)], hl_v, sem),
    ]
    for cp in cps:
        cp.start()
    for cp in cps:
        cp.wait()

    wv = w_v[pl.ds(0, L)]
    w1 = jnp.full((L,), wv[0], jnp.float32)
    w2 = jnp.full((L,), wv[1], jnp.float32)
    w3 = jnp.full((L,), wv[2], jnp.float32)
    lane = lax.iota(jnp.int32, L)

    for r in range(53):
        for off in (0, 8):
            wsc_v[pl.ds(r * DS + off, L)] = w1 * weeks_v[r, pl.ds(off, L)]

    for s in range(7):
        for h in range(2):
            r = (s * 2 + h) * DS
            for off in (0, 8):
                comb_v[pl.ds(r + off, L)] = (
                    w2 * seasons_v[s, pl.ds(off, L)]
                    + w3 * hol_v[h, pl.ds(off, L)]
                )

    dcols = []
    for t in range(D):
        x = lane + t
        dcols.append(jnp.where(x >= D, x - D, x))

    def group(g, carry):
        b0 = g * L
        wk = wk_v[pl.ds(b0, L)] * DS
        cb = dy_v[pl.ds(b0, L)] * (2 * DS) + hl_v[pl.ds(b0, L)] * DS
        rows = b0 + lane
        ilp = 6
        for t in range(D // ilp):
            ts = tuple(t + k * (D // ilp) for k in range(ilp))
            ab = [(plsc.load_gather(wsc_v, [wk + dcols[tt]]),
                   plsc.load_gather(comb_v, [cb + dcols[tt]])) for tt in ts]
            for tt, (a, b) in zip(ts, ab):
                plsc.store_scatter(stage_v, [rows, dcols[tt]], a + b)
        return carry

    out_cps = []
    for c in range(NCHUNK):
        lax.fori_loop(c * GPC, (c + 1) * GPC, group, 0)
        cp = pltpu.make_async_copy(
            stage_v.at[pl.ds(c * CROWS, CROWS)],
            out_hbm.at[pl.ds(base + c * CROWS, CROWS)],
            sem,
        )
        cp.start()
        out_cps.append(cp)
    for cp in out_cps:
        cp.wait()


def kernel(weeks, seasons, holidays_tab, w1, w2, w3, week_idx, day_idx, holiday_idx):
    w = jnp.stack([w1, w2, w3])
    mesh = plsc.VectorSubcoreMesh(core_axis_name="c", subcore_axis_name="s")
    f = pl.kernel(
        _sc_body,
        mesh=mesh,
        compiler_params=pltpu.CompilerParams(needs_layout_passes=False),
        out_type=jax.ShapeDtypeStruct((B, DP), jnp.float32),
        scratch_types=[
            pltpu.VMEM((53, D), jnp.float32),
            pltpu.VMEM((7, D), jnp.float32),
            pltpu.VMEM((2, D), jnp.float32),
            pltpu.VMEM((53 * DS + 8,), jnp.float32),
            pltpu.VMEM((14 * DS + 8,), jnp.float32),
            pltpu.VMEM((BPW,), jnp.int32),
            pltpu.VMEM((BPW,), jnp.int32),
            pltpu.VMEM((BPW,), jnp.int32),
            pltpu.VMEM((BPW, DP), jnp.float32),
            pltpu.VMEM((L,), jnp.float32),
            pltpu.SemaphoreType.DMA,
        ],
    )
    padded = f(weeks, seasons, holidays_tab, w,
               week_idx, day_idx, holiday_idx)
    return padded[:, :D]
